# baseline (speedup 1.0000x reference)
import functools

import jax
import jax.numpy as jnp
from jax import lax
from jax.experimental import pallas as pl
from jax.experimental.pallas import tpu as pltpu
from jax.experimental.pallas import tpu_sc as plsc

BATCH = 4096
SEQ = 200
EMB_DIM = 128
NUM_WORKERS = 32
ROWS_PER_WORKER = BATCH // NUM_WORKERS
HALF_SEQ = SEQ // 2
NCHUNK = EMB_DIM // 16


def _bag_body(text_hbm, emb_hbm, out_hbm, idx_v, rows_v, out_stage, sem0, sem1):
    wid = lax.axis_index("s") * 2 + lax.axis_index("c")
    base = wid * ROWS_PER_WORKER
    sems = (sem0, sem1)

    pltpu.sync_copy(text_hbm.at[pl.ds(base, ROWS_PER_WORKER)], idx_v)

    def issue(r, b):
        pltpu.async_copy(emb_hbm.at[idx_v.at[r, 0]], rows_v.at[b, 0], sems[b])
        pltpu.async_copy(emb_hbm.at[idx_v.at[r, 1]], rows_v.at[b, 1], sems[b])

    def wait(r, b):
        pltpu.make_async_copy(emb_hbm.at[idx_v.at[r, 0]], rows_v.at[b, 0], sems[b]).wait()
        pltpu.make_async_copy(emb_hbm.at[idx_v.at[r, 1]], rows_v.at[b, 1], sems[b]).wait()

    def accum(r, b):
        def tok_step(t, acc):
            for h in range(2):
                for u in range(2):
                    acc = tuple(
                        acc[c] + rows_v[b, h, 2 * t + u, pl.ds(c * 16, 16)]
                        for c in range(NCHUNK)
                    )
            return acc

        acc = tuple(jnp.zeros((16,), jnp.float32) for _ in range(NCHUNK))
        acc = lax.fori_loop(0, HALF_SEQ // 2, tok_step, acc)
        for c in range(NCHUNK):
            out_stage[r, pl.ds(c * 16, 16)] = acc[c]

    issue(0, 0)

    def body(g, _):
        r0 = 2 * g
        issue(r0 + 1, 1)
        wait(r0, 0)
        accum(r0, 0)

        @pl.when(g < ROWS_PER_WORKER // 2 - 1)
        def _():
            issue(r0 + 2, 0)

        wait(r0 + 1, 1)
        accum(r0 + 1, 1)
        return 0

    lax.fori_loop(0, ROWS_PER_WORKER // 2, body, 0)
    pltpu.sync_copy(out_stage, out_hbm.at[pl.ds(base, ROWS_PER_WORKER)])


def _embedding_bag(text3, emb):
    mesh = plsc.VectorSubcoreMesh(core_axis_name="c", subcore_axis_name="s")
    run = pl.kernel(
        _bag_body,
        out_type=jax.ShapeDtypeStruct((BATCH, EMB_DIM), jnp.float32),
        mesh=mesh,
        scratch_types=[
            pltpu.VMEM((ROWS_PER_WORKER, 2, HALF_SEQ), jnp.int32),
            pltpu.VMEM((2, 2, HALF_SEQ, EMB_DIM), jnp.float32),
            pltpu.VMEM((ROWS_PER_WORKER, EMB_DIM), jnp.float32),
            pltpu.SemaphoreType.DMA,
            pltpu.SemaphoreType.DMA,
        ],
    )
    return run(text3, emb)


def _mlp_body(x_ref, w1_ref, b1_ref, w2_ref, b2_ref, wc_ref, bc_ref, out_ref):
    x = jnp.tanh(x_ref[...])
    h1 = jnp.tanh(jnp.dot(x, w1_ref[...].T, preferred_element_type=jnp.float32) + b1_ref[...])
    h2 = jnp.tanh(jnp.dot(h1, w2_ref[...].T, preferred_element_type=jnp.float32) + b2_ref[...])
    logits = jnp.dot(h2, wc_ref[...].T, preferred_element_type=jnp.float32) + bc_ref[...]
    m = jnp.max(logits, axis=-1, keepdims=True)
    e = jnp.exp(logits - m)
    out_ref[...] = e / jnp.sum(e, axis=-1, keepdims=True)


def _mlp(summed, W1, b1, W2, b2, Wc, bc):
    blk = 512
    grid = (BATCH // blk,)
    full = lambda shape: pl.BlockSpec(shape, lambda i: (0,) * len(shape))
    return pl.pallas_call(
        _mlp_body,
        grid=grid,
        in_specs=[
            pl.BlockSpec((blk, EMB_DIM), lambda i: (i, 0)),
            full(W1.shape),
            full(b1.shape),
            full(W2.shape),
            full(b2.shape),
            full(Wc.shape),
            full(bc.shape),
        ],
        out_specs=pl.BlockSpec((blk, 2), lambda i: (i, 0)),
        out_shape=jax.ShapeDtypeStruct((BATCH, 2), jnp.float32),
    )(summed, W1, b1, W2, b2, Wc, bc)


def kernel(text, emb, W1, b1, W2, b2, Wc, bc):
    text3 = text.astype(jnp.int32).reshape(BATCH, 2, HALF_SEQ)
    summed = _embedding_bag(text3, emb)
    b1r = b1.reshape(1, -1)
    b2r = b2.reshape(1, -1)
    bcr = bc.reshape(1, -1)
    return _mlp(summed, W1, b1r, W2, b2r, Wc, bcr)


# speedup vs baseline: 1.2172x; 1.2172x over previous
import functools

import jax
import jax.numpy as jnp
from jax import lax
from jax.experimental import pallas as pl
from jax.experimental.pallas import tpu as pltpu
from jax.experimental.pallas import tpu_sc as plsc

BATCH = 4096
SEQ = 200
EMB_DIM = 128
NUM_WORKERS = 32
ROWS_PER_WORKER = BATCH // NUM_WORKERS
HALF_SEQ = SEQ // 2
NCHUNK = EMB_DIM // 16


def _bag_body(text_hbm, emb_hbm, out_hbm, idx_v, rows_v, out_stage, sem0, sem1, sem2):
    wid = lax.axis_index("s") * 2 + lax.axis_index("c")
    base = wid * ROWS_PER_WORKER
    sems = (sem0, sem1, sem2)

    pltpu.sync_copy(text_hbm.at[pl.ds(base, ROWS_PER_WORKER)], idx_v)

    def issue(r, b):
        pltpu.async_copy(emb_hbm.at[idx_v.at[r, 0]], rows_v.at[b, 0], sems[b])
        pltpu.async_copy(emb_hbm.at[idx_v.at[r, 1]], rows_v.at[b, 1], sems[b])

    def wait(r, b):
        pltpu.make_async_copy(emb_hbm.at[idx_v.at[r, 0]], rows_v.at[b, 0], sems[b]).wait()
        pltpu.make_async_copy(emb_hbm.at[idx_v.at[r, 1]], rows_v.at[b, 1], sems[b]).wait()

    def accum(r, b):
        def tok_step(t, acc):
            for h in range(2):
                for u in range(2):
                    acc = tuple(
                        acc[c] + rows_v[b, h, 2 * t + u, pl.ds(c * 16, 16)]
                        for c in range(NCHUNK)
                    )
            return acc

        acc = tuple(jnp.zeros((16,), jnp.float32) for _ in range(NCHUNK))
        acc = lax.fori_loop(0, HALF_SEQ // 2, tok_step, acc)
        for c in range(NCHUNK):
            out_stage[r, pl.ds(c * 16, 16)] = acc[c]

    issue(0, 0)
    issue(1, 1)
    issue(2, 2)

    def body(g, _):
        for b in range(3):
            r = 3 * g + b
            wait(r, b)
            accum(r, b)
            if b == 2:
                @pl.when(g < 41)
                def _():
                    issue(r + 3, b)
            else:
                issue(r + 3, b)
        return 0

    lax.fori_loop(0, 42, body, 0)
    wait(126, 0)
    accum(126, 0)
    wait(127, 1)
    accum(127, 1)
    pltpu.sync_copy(out_stage, out_hbm.at[pl.ds(base, ROWS_PER_WORKER)])


def _embedding_bag(text3, emb):
    mesh = plsc.VectorSubcoreMesh(core_axis_name="c", subcore_axis_name="s")
    run = pl.kernel(
        _bag_body,
        out_type=jax.ShapeDtypeStruct((BATCH, EMB_DIM), jnp.float32),
        mesh=mesh,
        scratch_types=[
            pltpu.VMEM((ROWS_PER_WORKER, 2, HALF_SEQ), jnp.int32),
            pltpu.VMEM((3, 2, HALF_SEQ, EMB_DIM), jnp.float32),
            pltpu.VMEM((ROWS_PER_WORKER, EMB_DIM), jnp.float32),
            pltpu.SemaphoreType.DMA,
            pltpu.SemaphoreType.DMA,
            pltpu.SemaphoreType.DMA,
        ],
    )
    return run(text3, emb)


def _mlp_body(x_ref, w1_ref, b1_ref, w2_ref, b2_ref, wc_ref, bc_ref, out_ref):
    x = jnp.tanh(x_ref[...])
    h1 = jnp.tanh(jnp.dot(x, w1_ref[...].T, preferred_element_type=jnp.float32) + b1_ref[...])
    h2 = jnp.tanh(jnp.dot(h1, w2_ref[...].T, preferred_element_type=jnp.float32) + b2_ref[...])
    logits = jnp.dot(h2, wc_ref[...].T, preferred_element_type=jnp.float32) + bc_ref[...]
    m = jnp.max(logits, axis=-1, keepdims=True)
    e = jnp.exp(logits - m)
    out_ref[...] = e / jnp.sum(e, axis=-1, keepdims=True)


def _mlp(summed, W1, b1, W2, b2, Wc, bc):
    blk = 512
    grid = (BATCH // blk,)
    full = lambda shape: pl.BlockSpec(shape, lambda i: (0,) * len(shape))
    return pl.pallas_call(
        _mlp_body,
        grid=grid,
        in_specs=[
            pl.BlockSpec((blk, EMB_DIM), lambda i: (i, 0)),
            full(W1.shape),
            full(b1.shape),
            full(W2.shape),
            full(b2.shape),
            full(Wc.shape),
            full(bc.shape),
        ],
        out_specs=pl.BlockSpec((blk, 2), lambda i: (i, 0)),
        out_shape=jax.ShapeDtypeStruct((BATCH, 2), jnp.float32),
    )(summed, W1, b1, W2, b2, Wc, bc)


def kernel(text, emb, W1, b1, W2, b2, Wc, bc):
    text3 = text.astype(jnp.int32).reshape(BATCH, 2, HALF_SEQ)
    summed = _embedding_bag(text3, emb)
    b1r = b1.reshape(1, -1)
    b2r = b2.reshape(1, -1)
    bcr = bc.reshape(1, -1)
    return _mlp(summed, W1, b1r, W2, b2r, Wc, bcr)
